# baseline (speedup 1.0000x reference)
import jax
import jax.numpy as jnp
from jax.experimental import pallas as pl
from jax.experimental.pallas import tpu as pltpu

R = 128
C = 100000
BA = 2048
BW = 2048
NBA = (C + BA - 1) // BA
NBW = (C + BW - 1) // BW
EPS = 1e-10


def _threefry_bits(flat_i32):
    u32 = jnp.uint32
    ks0 = u32(0)
    ks1 = u32(1)
    ks2 = u32(0x1BD11BDA) ^ ks0 ^ ks1
    x1 = flat_i32.astype(jnp.uint32) + ks1
    x0 = jnp.zeros_like(x1) + ks0

    def rotl(x, r):
        return (x << u32(r)) | (x >> u32(32 - r))

    rots = ((13, 15, 26, 6), (17, 29, 16, 24))
    adds = ((ks1, ks2 + u32(1)), (ks2, ks0 + u32(2)), (ks0, ks1 + u32(3)),
            (ks1, ks2 + u32(4)), (ks2, ks0 + u32(5)))
    for g in range(5):
        for r in rots[g % 2]:
            x0 = x0 + x1
            x1 = rotl(x1, r)
            x1 = x1 ^ x0
        x0 = x0 + adds[g][0]
        x1 = x1 + adds[g][1]
    return x0 ^ x1


def _gumbel(col_global):
    row = jax.lax.broadcasted_iota(jnp.int32, col_global.shape, 0)
    flat = row * C + col_global
    bits = _threefry_bits(flat)
    fb = (bits >> jnp.uint32(9)) | jnp.uint32(0x3F800000)
    u = jax.lax.bitcast_convert_type(fb, jnp.float32) - jnp.float32(1.0)
    return -jnp.log(jnp.float32(EPS) - jnp.log(u + jnp.float32(EPS)))


def _argmax_kernel(x_ref, idx_ref, m_scr, i_scr):
    j = pl.program_id(0)
    col = jax.lax.broadcasted_iota(jnp.int32, (R, BA), 1) + j * BA
    z = x_ref[...] + _gumbel(col)
    z = jnp.where(col < C, z, -jnp.inf)

    @pl.when(j == 0)
    def _init():
        m_scr[...] = z
        i_scr[...] = col

    @pl.when(j > 0)
    def _update():
        m = m_scr[...]
        better = z > m
        m_scr[...] = jnp.where(better, z, m)
        i_scr[...] = jnp.where(better, col, i_scr[...])

    @pl.when(j == NBA - 1)
    def _finish():
        m = m_scr[...]
        best = jnp.max(m, axis=1, keepdims=True)
        idx = jnp.min(jnp.where(m == best, i_scr[...], jnp.int32(C)),
                      axis=1, keepdims=True)
        idx_ref[...] = jnp.broadcast_to(idx, (R, 128))


def _onehot_kernel(idx_ref, o_ref):
    j = pl.program_id(0)
    col = jax.lax.broadcasted_iota(jnp.int32, (R, BW), 1) + j * BW
    o_ref[...] = jnp.where(col == idx_ref[:, 0:1], jnp.float32(1.0),
                           jnp.float32(0.0))


def kernel(logits):
    idx = pl.pallas_call(
        _argmax_kernel,
        grid=(NBA,),
        in_specs=[pl.BlockSpec((R, BA), lambda j: (0, j))],
        out_specs=pl.BlockSpec((R, 128), lambda j: (0, 0)),
        out_shape=jax.ShapeDtypeStruct((R, 128), jnp.int32),
        scratch_shapes=[pltpu.VMEM((R, BA), jnp.float32),
                        pltpu.VMEM((R, BA), jnp.int32)],
        compiler_params=pltpu.CompilerParams(
            dimension_semantics=("arbitrary",)),
    )(logits)
    return pl.pallas_call(
        _onehot_kernel,
        grid=(NBW,),
        in_specs=[pl.BlockSpec((R, 128), lambda j: (0, 0))],
        out_specs=pl.BlockSpec((R, BW), lambda j: (0, j)),
        out_shape=jax.ShapeDtypeStruct((R, C), jnp.float32),
        compiler_params=pltpu.CompilerParams(
            dimension_semantics=("arbitrary",)),
    )(idx)


# speedup vs baseline: 1.0586x; 1.0586x over previous
import jax
import jax.numpy as jnp
from jax.experimental import pallas as pl
from jax.experimental.pallas import tpu as pltpu

R = 128
C = 100000
RB = 8
EPS = 1e-10


def _threefry_bits(flat_i32):
    u32 = jnp.uint32
    ks0 = u32(0)
    ks1 = u32(1)
    ks2 = u32(0x1BD11BDA) ^ ks0 ^ ks1
    x1 = flat_i32.astype(jnp.uint32) + ks1
    x0 = jnp.zeros_like(x1) + ks0

    def rotl(x, r):
        return (x << u32(r)) | (x >> u32(32 - r))

    rots = ((13, 15, 26, 6), (17, 29, 16, 24))
    adds = ((ks1, ks2 + u32(1)), (ks2, ks0 + u32(2)), (ks0, ks1 + u32(3)),
            (ks1, ks2 + u32(4)), (ks2, ks0 + u32(5)))
    for g in range(5):
        for r in rots[g % 2]:
            x0 = x0 + x1
            x1 = rotl(x1, r)
            x1 = x1 ^ x0
        x0 = x0 + adds[g][0]
        x1 = x1 + adds[g][1]
    return x0 ^ x1


def _fused_kernel(x_ref, o_ref):
    i = pl.program_id(0)
    col = jax.lax.broadcasted_iota(jnp.int32, (RB, C), 1)
    row = jax.lax.broadcasted_iota(jnp.int32, (RB, 1), 0) + i * RB
    flat = row * C + col
    bits = _threefry_bits(flat)
    fb = (bits >> jnp.uint32(9)) | jnp.uint32(0x3F800000)
    u = jax.lax.bitcast_convert_type(fb, jnp.float32) - jnp.float32(1.0)
    g = -jnp.log(jnp.float32(EPS) - jnp.log(u + jnp.float32(EPS)))
    z = x_ref[...] + g
    best = jnp.max(z, axis=1, keepdims=True)
    idx = jnp.min(jnp.where(z == best, col, jnp.int32(C)), axis=1,
                  keepdims=True)
    o_ref[...] = jnp.where(col == idx, jnp.float32(1.0), jnp.float32(0.0))


def kernel(logits):
    return pl.pallas_call(
        _fused_kernel,
        grid=(R // RB,),
        in_specs=[pl.BlockSpec((RB, C), lambda i: (i, 0))],
        out_specs=pl.BlockSpec((RB, C), lambda i: (i, 0)),
        out_shape=jax.ShapeDtypeStruct((R, C), jnp.float32),
        compiler_params=pltpu.CompilerParams(
            dimension_semantics=("arbitrary",)),
    )(logits)
